# baseline (speedup 1.0000x reference)
import functools

import jax
import jax.numpy as jnp
from jax import lax
from jax.experimental import pallas as pl
from jax.experimental.pallas import tpu as pltpu
from jax.experimental.pallas import tpu_sc as plsc

_B = 8
_K = 16
_C = 3
_H = 384
_W = 384
_P = _H * _W
_NW = 32
_WPB = _NW // _B
_QW = _P // _WPB
_CH = 12288
_NCH = _QW // _CH
_LANES = 16
_U = 1
_ALPHA = 0.001
_NPAIR = _K * (_K - 1) / 2.0


def _rsqrt(s):
    i = lax.bitcast_convert_type(s, jnp.int32)
    i = 0x5F3759DF - lax.shift_right_arithmetic(i, 1)
    y = lax.bitcast_convert_type(i, jnp.float32)
    for _ in range(3):
        y = y * (1.5 - 0.5 * s * y * y)
    return y


def _sc_body(img, palb, palv, out_px, out_pal, b0r, b0g, b0b, b1r, b1g, b1b,
             mbuf, palb_v, pald_v, palv_v, stage_v, sem0, sem1, semp):
    cid = lax.axis_index("c")
    sid = lax.axis_index("s")
    wid = sid * 2 + cid
    b = wid // _WPB
    q = wid % _WPB

    bufs = [(b0r, b0g, b0b), (b1r, b1g, b1b)]
    sems = [sem0, sem1]
    base = (b * _C) * _P + q * _QW

    def start_chunk(ch, s):
        off = base + ch * _CH
        return [
            pltpu.async_copy(img.at[pl.ds(off + c * _P, _CH)], bufs[s][c],
                             sems[s])
            for c in range(_C)
        ]

    handles = start_chunk(0, 0)
    palv_h = pltpu.async_copy(palv.at[b], palv_v, semp)

    pltpu.sync_copy(palb.at[b], palb_v)
    pr = [palb_v[pl.ds((0 * _K + k) * _LANES, _LANES)] for k in range(_K)]
    pg = [palb_v[pl.ds((1 * _K + k) * _LANES, _LANES)] for k in range(_K)]
    pb = [palb_v[pl.ds((2 * _K + k) * _LANES, _LANES)] for k in range(_K)]

    for k in range(_K):
        pal_k = (k * 4) * _LANES
        pald_v[pl.ds(pal_k + 0 * _LANES, _LANES)] = pr[k] + pr[k]
        pald_v[pl.ds(pal_k + 1 * _LANES, _LANES)] = pg[k] + pg[k]
        pald_v[pl.ds(pal_k + 2 * _LANES, _LANES)] = pb[k] + pb[k]
        pald_v[pl.ds(pal_k + 3 * _LANES, _LANES)] = (
            pr[k] * pr[k] + pg[k] * pg[k] + pb[k] * pb[k])

    def _dot_vecs(k):
        pal_k = (k * 4) * _LANES
        return (pald_v[pl.ds(pal_k + 0 * _LANES, _LANES)],
                pald_v[pl.ds(pal_k + 1 * _LANES, _LANES)],
                pald_v[pl.ds(pal_k + 2 * _LANES, _LANES)],
                pald_v[pl.ds(pal_k + 3 * _LANES, _LANES)])

    def chunk_compute(s, acc, acc2, mbuf):
        br, bg, bb = bufs[s]

        da = [_dot_vecs(k) for k in range(_K // 2)]

        def body_a(i, acc2):
            for v in range(2):
                off = i * (2 * _LANES) + v * _LANES
                r = br[pl.ds(off, _LANES)]
                g = bg[pl.ds(off, _LANES)]
                bl = bb[pl.ds(off, _LANES)]
                m = None
                for ak, bk, gk, ck in da:
                    t = (ck - r * ak) - (g * bk + bl * gk)
                    m = t if m is None else jnp.minimum(m, t)
                mbuf[pl.ds(off, _LANES)] = m
                acc2 = acc2 + (r * r + g * g + bl * bl)
            return acc2

        acc2 = lax.fori_loop(0, _CH // (2 * _LANES), body_a, acc2)

        db2 = [_dot_vecs(k) for k in range(_K // 2, _K)]

        def body_b(i, acc):
            for v in range(2):
                off = i * (2 * _LANES) + v * _LANES
                r = br[pl.ds(off, _LANES)]
                g = bg[pl.ds(off, _LANES)]
                bl = bb[pl.ds(off, _LANES)]
                m = mbuf[pl.ds(off, _LANES)]
                for ak, bk, gk, ck in db2:
                    t = (ck - r * ak) - (g * bk + bl * gk)
                    m = jnp.minimum(m, t)
                acc = acc + m
            return acc

        return lax.fori_loop(0, _CH // (2 * _LANES), body_b, acc), acc2

    acc = jnp.zeros((_LANES,), jnp.float32)
    acc2 = jnp.zeros((_LANES,), jnp.float32)
    palv_h.wait()
    for ch in range(_NCH):
        s = ch % 2
        for h in handles:
            h.wait()
        if ch + 1 < _NCH:
            handles = start_chunk(ch + 1, (ch + 1) % 2)
        acc, acc2 = chunk_compute(s, acc, acc2, mbuf)

    stage_v[...] = acc + acc2
    pltpu.sync_copy(stage_v, out_px.at[wid])

    @pl.when(q == 0)
    def _():
        lanes = lax.iota(jnp.int32, _LANES)
        pv0 = palv_v[pl.ds(0 * _LANES, _LANES)]
        pv1 = palv_v[pl.ds(1 * _LANES, _LANES)]
        pv2 = palv_v[pl.ds(2 * _LANES, _LANES)]
        pal_acc = jnp.zeros((_LANES,), jnp.float32)
        for j in range(_K):
            dr = pv0 - pr[j]
            dg = pv1 - pg[j]
            db = pv2 - pb[j]
            d2 = dr * dr + dg * dg + db * db
            dist = d2 * _rsqrt(d2)
            mask = jnp.where(lanes > j, 1.0, 0.0).astype(jnp.float32)
            pal_acc = pal_acc + dist * mask
        stage_v[...] = pal_acc
        pltpu.sync_copy(stage_v, out_pal.at[b])


_sc_kernel = functools.partial(
    pl.kernel,
    out_type=[
        jax.ShapeDtypeStruct((_NW, _LANES), jnp.float32),
        jax.ShapeDtypeStruct((_B, _LANES), jnp.float32),
    ],
    mesh=plsc.VectorSubcoreMesh(core_axis_name="c", subcore_axis_name="s"),
    scratch_types=[
        pltpu.VMEM((_CH,), jnp.float32),
        pltpu.VMEM((_CH,), jnp.float32),
        pltpu.VMEM((_CH,), jnp.float32),
        pltpu.VMEM((_CH,), jnp.float32),
        pltpu.VMEM((_CH,), jnp.float32),
        pltpu.VMEM((_CH,), jnp.float32),
        pltpu.VMEM((_CH,), jnp.float32),
        pltpu.VMEM((_C * _K * _LANES,), jnp.float32),
        pltpu.VMEM((4 * _K * _LANES,), jnp.float32),
        pltpu.VMEM((_C * _LANES,), jnp.float32),
        pltpu.VMEM((_LANES,), jnp.float32),
        pltpu.SemaphoreType.DMA,
        pltpu.SemaphoreType.DMA,
        pltpu.SemaphoreType.DMA,
    ],
)(_sc_body)


@jax.jit
def kernel(palettes, images):
    palv = jnp.transpose(palettes, (0, 2, 1))
    palb = jnp.broadcast_to(palv[..., None], (_B, _C, _K, _LANES))
    palb = palb.reshape(_B, _C * _K * _LANES)
    img = images.reshape(_B * _C * _P)
    out_px, out_pal = _sc_kernel(img, palb, palv.reshape(_B, _C * _K))
    mse = jnp.sum(out_px) / (_B * _C * _P)
    pal = jnp.sum(out_pal) / (_NPAIR * _B)
    return mse - _ALPHA * pal


# speedup vs baseline: 1.0206x; 1.0206x over previous
import functools

import jax
import jax.numpy as jnp
from jax import lax
from jax.experimental import pallas as pl
from jax.experimental.pallas import tpu as pltpu
from jax.experimental.pallas import tpu_sc as plsc

_B = 8
_K = 16
_C = 3
_H = 384
_W = 384
_P = _H * _W
_NW = 32
_WPB = _NW // _B
_QW = _P // _WPB
_CH = 12288
_NCH = _QW // _CH
_LANES = 16
_U = 1
_ALPHA = 0.001
_NPAIR = _K * (_K - 1) / 2.0


def _rsqrt(s):
    i = lax.bitcast_convert_type(s, jnp.int32)
    i = 0x5F3759DF - lax.shift_right_arithmetic(i, 1)
    y = lax.bitcast_convert_type(i, jnp.float32)
    for _ in range(3):
        y = y * (1.5 - 0.5 * s * y * y)
    return y


def _sc_body(img, palb, palv, out_px, out_pal, b0r, b0g, b0b, b1r, b1g, b1b,
             mbuf, palb_v, pald_v, palv_v, stage_v, sem0, sem1, semp):
    cid = lax.axis_index("c")
    sid = lax.axis_index("s")
    wid = sid * 2 + cid
    b = wid // _WPB
    q = wid % _WPB

    bufs = [(b0r, b0g, b0b), (b1r, b1g, b1b)]
    sems = [sem0, sem1]
    base = (b * _C) * _P + q * _QW

    def start_chunk(ch, s):
        off = base + ch * _CH
        return [
            pltpu.async_copy(img.at[pl.ds(off + c * _P, _CH)], bufs[s][c],
                             sems[s])
            for c in range(_C)
        ]

    handles = start_chunk(0, 0)
    palv_h = pltpu.async_copy(palv.at[b], palv_v, semp)

    pltpu.sync_copy(palb.at[b], palb_v)
    pr = [palb_v[pl.ds((0 * _K + k) * _LANES, _LANES)] for k in range(_K)]
    pg = [palb_v[pl.ds((1 * _K + k) * _LANES, _LANES)] for k in range(_K)]
    pb = [palb_v[pl.ds((2 * _K + k) * _LANES, _LANES)] for k in range(_K)]

    for k in range(_K):
        pal_k = (k * 4) * _LANES
        pald_v[pl.ds(pal_k + 0 * _LANES, _LANES)] = pr[k] + pr[k]
        pald_v[pl.ds(pal_k + 1 * _LANES, _LANES)] = pg[k] + pg[k]
        pald_v[pl.ds(pal_k + 2 * _LANES, _LANES)] = pb[k] + pb[k]
        pald_v[pl.ds(pal_k + 3 * _LANES, _LANES)] = (
            pr[k] * pr[k] + pg[k] * pg[k] + pb[k] * pb[k])

    def _dot_vecs(k):
        pal_k = (k * 4) * _LANES
        return (pald_v[pl.ds(pal_k + 0 * _LANES, _LANES)],
                pald_v[pl.ds(pal_k + 1 * _LANES, _LANES)],
                pald_v[pl.ds(pal_k + 2 * _LANES, _LANES)],
                pald_v[pl.ds(pal_k + 3 * _LANES, _LANES)])

    def chunk_compute(s, acc, acc2, mbuf):
        br, bg, bb = bufs[s]

        da = [_dot_vecs(k) for k in range(_K // 2)]

        def body_a(i, acc2):
            off = i * _LANES
            r = br[pl.ds(off, _LANES)]
            g = bg[pl.ds(off, _LANES)]
            bl = bb[pl.ds(off, _LANES)]
            m = None
            for ak, bk, gk, ck in da:
                t = (ck - r * ak) - (g * bk + bl * gk)
                m = t if m is None else jnp.minimum(m, t)
            mbuf[pl.ds(off, _LANES)] = m
            return acc2 + (r * r + g * g + bl * bl)

        acc2 = lax.fori_loop(0, _CH // _LANES, body_a, acc2)

        db2 = [_dot_vecs(k) for k in range(_K // 2, _K)]

        def body_b(i, acc):
            off = i * _LANES
            r = br[pl.ds(off, _LANES)]
            g = bg[pl.ds(off, _LANES)]
            bl = bb[pl.ds(off, _LANES)]
            m = mbuf[pl.ds(off, _LANES)]
            for ak, bk, gk, ck in db2:
                t = (ck - r * ak) - (g * bk + bl * gk)
                m = jnp.minimum(m, t)
            return acc + m

        return lax.fori_loop(0, _CH // _LANES, body_b, acc), acc2

    acc = jnp.zeros((_LANES,), jnp.float32)
    acc2 = jnp.zeros((_LANES,), jnp.float32)
    for ch in range(_NCH):
        s = ch % 2
        for h in handles:
            h.wait()
        if ch + 1 < _NCH:
            handles = start_chunk(ch + 1, (ch + 1) % 2)
        acc, acc2 = chunk_compute(s, acc, acc2, mbuf)

    stage_v[...] = acc + acc2
    pltpu.sync_copy(stage_v, out_px.at[wid])

    @pl.when(q == 0)
    def _():
        palv_h.wait()
        lanes = lax.iota(jnp.int32, _LANES)
        pv0 = palv_v[pl.ds(0 * _LANES, _LANES)]
        pv1 = palv_v[pl.ds(1 * _LANES, _LANES)]
        pv2 = palv_v[pl.ds(2 * _LANES, _LANES)]
        pal_acc = jnp.zeros((_LANES,), jnp.float32)
        for j in range(_K):
            dr = pv0 - pr[j]
            dg = pv1 - pg[j]
            db = pv2 - pb[j]
            d2 = dr * dr + dg * dg + db * db
            dist = d2 * _rsqrt(d2)
            mask = jnp.where(lanes > j, 1.0, 0.0).astype(jnp.float32)
            pal_acc = pal_acc + dist * mask
        stage_v[...] = pal_acc
        pltpu.sync_copy(stage_v, out_pal.at[b])


_sc_kernel = functools.partial(
    pl.kernel,
    out_type=[
        jax.ShapeDtypeStruct((_NW, _LANES), jnp.float32),
        jax.ShapeDtypeStruct((_B, _LANES), jnp.float32),
    ],
    mesh=plsc.VectorSubcoreMesh(core_axis_name="c", subcore_axis_name="s"),
    scratch_types=[
        pltpu.VMEM((_CH,), jnp.float32),
        pltpu.VMEM((_CH,), jnp.float32),
        pltpu.VMEM((_CH,), jnp.float32),
        pltpu.VMEM((_CH,), jnp.float32),
        pltpu.VMEM((_CH,), jnp.float32),
        pltpu.VMEM((_CH,), jnp.float32),
        pltpu.VMEM((_CH,), jnp.float32),
        pltpu.VMEM((_C * _K * _LANES,), jnp.float32),
        pltpu.VMEM((4 * _K * _LANES,), jnp.float32),
        pltpu.VMEM((_C * _LANES,), jnp.float32),
        pltpu.VMEM((_LANES,), jnp.float32),
        pltpu.SemaphoreType.DMA,
        pltpu.SemaphoreType.DMA,
        pltpu.SemaphoreType.DMA,
    ],
)(_sc_body)


@jax.jit
def kernel(palettes, images):
    palv = jnp.transpose(palettes, (0, 2, 1))
    palb = jnp.broadcast_to(palv[..., None], (_B, _C, _K, _LANES))
    palb = palb.reshape(_B, _C * _K * _LANES)
    img = images.reshape(_B * _C * _P)
    out_px, out_pal = _sc_kernel(img, palb, palv.reshape(_B, _C * _K))
    mse = jnp.sum(out_px) / (_B * _C * _P)
    pal = jnp.sum(out_pal) / (_NPAIR * _B)
    return mse - _ALPHA * pal


# speedup vs baseline: 1.0215x; 1.0008x over previous
import functools

import jax
import jax.numpy as jnp
from jax import lax
from jax.experimental import pallas as pl
from jax.experimental.pallas import tpu as pltpu
from jax.experimental.pallas import tpu_sc as plsc

_B = 8
_K = 16
_C = 3
_H = 384
_W = 384
_P = _H * _W
_NW = 32
_WPB = _NW // _B
_QW = _P // _WPB
_CH = 12288
_NCH = _QW // _CH
_LANES = 16
_U = 1
_ALPHA = 0.001
_NPAIR = _K * (_K - 1) / 2.0


def _rsqrt(s):
    i = lax.bitcast_convert_type(s, jnp.int32)
    i = 0x5F3759DF - lax.shift_right_arithmetic(i, 1)
    y = lax.bitcast_convert_type(i, jnp.float32)
    for _ in range(3):
        y = y * (1.5 - 0.5 * s * y * y)
    return y


def _sc_body(img, palb, palv, out_px, out_pal, b0r, b0g, b0b, b1r, b1g, b1b,
             mbuf, palb_v, pald_v, palv_v, stage_v, sem0, sem1, semp):
    cid = lax.axis_index("c")
    sid = lax.axis_index("s")
    wid = sid * 2 + cid
    b = wid // _WPB
    q = wid % _WPB

    bufs = [(b0r, b0g, b0b), (b1r, b1g, b1b)]
    sems = [sem0, sem1]
    base = (b * _C) * _P + q * _QW

    def start_chunk(ch, s):
        off = base + ch * _CH
        return [
            pltpu.async_copy(img.at[pl.ds(off + c * _P, _CH)], bufs[s][c],
                             sems[s])
            for c in range(_C)
        ]

    handles = start_chunk(0, 0)
    palv_h = pltpu.async_copy(palv.at[b], palv_v, semp)

    pltpu.sync_copy(palb.at[b], palb_v)
    pr = [palb_v[pl.ds((0 * _K + k) * _LANES, _LANES)] for k in range(_K)]
    pg = [palb_v[pl.ds((1 * _K + k) * _LANES, _LANES)] for k in range(_K)]
    pb = [palb_v[pl.ds((2 * _K + k) * _LANES, _LANES)] for k in range(_K)]

    for k in range(_K):
        pal_k = (k * 4) * _LANES
        pald_v[pl.ds(pal_k + 0 * _LANES, _LANES)] = pr[k] + pr[k]
        pald_v[pl.ds(pal_k + 1 * _LANES, _LANES)] = pg[k] + pg[k]
        pald_v[pl.ds(pal_k + 2 * _LANES, _LANES)] = pb[k] + pb[k]
        pald_v[pl.ds(pal_k + 3 * _LANES, _LANES)] = (
            pr[k] * pr[k] + pg[k] * pg[k] + pb[k] * pb[k])

    def _dot_vecs(k):
        pal_k = (k * 4) * _LANES
        return (pald_v[pl.ds(pal_k + 0 * _LANES, _LANES)],
                pald_v[pl.ds(pal_k + 1 * _LANES, _LANES)],
                pald_v[pl.ds(pal_k + 2 * _LANES, _LANES)],
                pald_v[pl.ds(pal_k + 3 * _LANES, _LANES)])

    def chunk_compute(s, acc, acc2, mbuf):
        br, bg, bb = bufs[s]

        da = [_dot_vecs(k) for k in range(_K // 2)]

        def body_a(i, acc2):
            off = i * _LANES
            r = br[pl.ds(off, _LANES)]
            g = bg[pl.ds(off, _LANES)]
            bl = bb[pl.ds(off, _LANES)]
            m = None
            for ak, bk, gk, ck in da:
                t = (ck - r * ak) - (g * bk + bl * gk)
                m = t if m is None else jnp.minimum(m, t)
            mbuf[pl.ds(off, _LANES)] = m
            return acc2 + (r * r + g * g + bl * bl)

        acc2 = plsc.parallel_loop(0, _CH // _LANES, unroll=2,
                                  carry=acc2)(body_a)

        db2 = [_dot_vecs(k) for k in range(_K // 2, _K)]

        def body_b(i, acc):
            off = i * _LANES
            r = br[pl.ds(off, _LANES)]
            g = bg[pl.ds(off, _LANES)]
            bl = bb[pl.ds(off, _LANES)]
            m = mbuf[pl.ds(off, _LANES)]
            for ak, bk, gk, ck in db2:
                t = (ck - r * ak) - (g * bk + bl * gk)
                m = jnp.minimum(m, t)
            return acc + m

        return plsc.parallel_loop(0, _CH // _LANES, unroll=2,
                                  carry=acc)(body_b), acc2

    acc = jnp.zeros((_LANES,), jnp.float32)
    acc2 = jnp.zeros((_LANES,), jnp.float32)
    for ch in range(_NCH):
        s = ch % 2
        for h in handles:
            h.wait()
        if ch + 1 < _NCH:
            handles = start_chunk(ch + 1, (ch + 1) % 2)
        acc, acc2 = chunk_compute(s, acc, acc2, mbuf)

    stage_v[...] = acc + acc2
    pltpu.sync_copy(stage_v, out_px.at[wid])

    @pl.when(q == 0)
    def _():
        palv_h.wait()
        lanes = lax.iota(jnp.int32, _LANES)
        pv0 = palv_v[pl.ds(0 * _LANES, _LANES)]
        pv1 = palv_v[pl.ds(1 * _LANES, _LANES)]
        pv2 = palv_v[pl.ds(2 * _LANES, _LANES)]
        pal_acc = jnp.zeros((_LANES,), jnp.float32)
        for j in range(_K):
            dr = pv0 - pr[j]
            dg = pv1 - pg[j]
            db = pv2 - pb[j]
            d2 = dr * dr + dg * dg + db * db
            dist = d2 * _rsqrt(d2)
            mask = jnp.where(lanes > j, 1.0, 0.0).astype(jnp.float32)
            pal_acc = pal_acc + dist * mask
        stage_v[...] = pal_acc
        pltpu.sync_copy(stage_v, out_pal.at[b])


_sc_kernel = functools.partial(
    pl.kernel,
    out_type=[
        jax.ShapeDtypeStruct((_NW, _LANES), jnp.float32),
        jax.ShapeDtypeStruct((_B, _LANES), jnp.float32),
    ],
    mesh=plsc.VectorSubcoreMesh(core_axis_name="c", subcore_axis_name="s"),
    scratch_types=[
        pltpu.VMEM((_CH,), jnp.float32),
        pltpu.VMEM((_CH,), jnp.float32),
        pltpu.VMEM((_CH,), jnp.float32),
        pltpu.VMEM((_CH,), jnp.float32),
        pltpu.VMEM((_CH,), jnp.float32),
        pltpu.VMEM((_CH,), jnp.float32),
        pltpu.VMEM((_CH,), jnp.float32),
        pltpu.VMEM((_C * _K * _LANES,), jnp.float32),
        pltpu.VMEM((4 * _K * _LANES,), jnp.float32),
        pltpu.VMEM((_C * _LANES,), jnp.float32),
        pltpu.VMEM((_LANES,), jnp.float32),
        pltpu.SemaphoreType.DMA,
        pltpu.SemaphoreType.DMA,
        pltpu.SemaphoreType.DMA,
    ],
)(_sc_body)


@jax.jit
def kernel(palettes, images):
    palv = jnp.transpose(palettes, (0, 2, 1))
    palb = jnp.broadcast_to(palv[..., None], (_B, _C, _K, _LANES))
    palb = palb.reshape(_B, _C * _K * _LANES)
    img = images.reshape(_B * _C * _P)
    out_px, out_pal = _sc_kernel(img, palb, palv.reshape(_B, _C * _K))
    mse = jnp.sum(out_px) / (_B * _C * _P)
    pal = jnp.sum(out_pal) / (_NPAIR * _B)
    return mse - _ALPHA * pal


# speedup vs baseline: 1.0227x; 1.0012x over previous
import functools

import jax
import jax.numpy as jnp
from jax import lax
from jax.experimental import pallas as pl
from jax.experimental.pallas import tpu as pltpu
from jax.experimental.pallas import tpu_sc as plsc

_B = 8
_K = 16
_C = 3
_H = 384
_W = 384
_P = _H * _W
_NW = 32
_WPB = _NW // _B
_QW = _P // _WPB
_CH = 12288
_NCH = _QW // _CH
_LANES = 16
_U = 1
_ALPHA = 0.001
_NPAIR = _K * (_K - 1) / 2.0


def _rsqrt(s):
    i = lax.bitcast_convert_type(s, jnp.int32)
    i = 0x5F3759DF - lax.shift_right_arithmetic(i, 1)
    y = lax.bitcast_convert_type(i, jnp.float32)
    for _ in range(3):
        y = y * (1.5 - 0.5 * s * y * y)
    return y


def _sc_body(img, palb, palv, out_px, out_pal, b0r, b0g, b0b, b1r, b1g, b1b,
             mbuf, palb_v, pald_v, palv_v, stage_v, sem0, sem1, semp):
    cid = lax.axis_index("c")
    sid = lax.axis_index("s")
    wid = sid * 2 + cid
    b = wid // _WPB
    q = wid % _WPB

    bufs = [(b0r, b0g, b0b), (b1r, b1g, b1b)]
    sems = [sem0, sem1]
    base = (b * _C) * _P + q * _QW

    def start_chunk(ch, s):
        off = base + ch * _CH
        return [
            pltpu.async_copy(img.at[pl.ds(off + c * _P, _CH)], bufs[s][c],
                             sems[s])
            for c in range(_C)
        ]

    handles = start_chunk(0, 0)
    palv_h = pltpu.async_copy(palv.at[b], palv_v, semp)

    pltpu.sync_copy(palb.at[b], palb_v)
    pr = [palb_v[pl.ds((0 * _K + k) * _LANES, _LANES)] for k in range(_K)]
    pg = [palb_v[pl.ds((1 * _K + k) * _LANES, _LANES)] for k in range(_K)]
    pb = [palb_v[pl.ds((2 * _K + k) * _LANES, _LANES)] for k in range(_K)]

    for k in range(_K):
        pal_k = (k * 4) * _LANES
        pald_v[pl.ds(pal_k + 0 * _LANES, _LANES)] = pr[k] + pr[k]
        pald_v[pl.ds(pal_k + 1 * _LANES, _LANES)] = pg[k] + pg[k]
        pald_v[pl.ds(pal_k + 2 * _LANES, _LANES)] = pb[k] + pb[k]
        pald_v[pl.ds(pal_k + 3 * _LANES, _LANES)] = (
            pr[k] * pr[k] + pg[k] * pg[k] + pb[k] * pb[k])

    def _dot_vecs(k):
        pal_k = (k * 4) * _LANES
        return (pald_v[pl.ds(pal_k + 0 * _LANES, _LANES)],
                pald_v[pl.ds(pal_k + 1 * _LANES, _LANES)],
                pald_v[pl.ds(pal_k + 2 * _LANES, _LANES)],
                pald_v[pl.ds(pal_k + 3 * _LANES, _LANES)])

    def chunk_compute(s, acc, acc2, mbuf):
        br, bg, bb = bufs[s]

        da = [_dot_vecs(k) for k in range(_K // 2)]

        def body_a(i, acc2):
            off = i * _LANES
            r = br[pl.ds(off, _LANES)]
            g = bg[pl.ds(off, _LANES)]
            bl = bb[pl.ds(off, _LANES)]
            m = None
            for ak, bk, gk, ck in da:
                t = (ck - r * ak) - (g * bk + bl * gk)
                m = t if m is None else jnp.minimum(m, t)
            mbuf[pl.ds(off, _LANES)] = m
            return acc2 + (r * r + g * g + bl * bl)

        acc2 = plsc.parallel_loop(0, _CH // _LANES, unroll=2,
                                  carry=acc2)(body_a)

        db2 = [_dot_vecs(k) for k in range(_K // 2, _K)]

        def body_b(i, acc):
            off = i * _LANES
            r = br[pl.ds(off, _LANES)]
            g = bg[pl.ds(off, _LANES)]
            bl = bb[pl.ds(off, _LANES)]
            m = mbuf[pl.ds(off, _LANES)]
            for ak, bk, gk, ck in db2:
                t = (ck - r * ak) - (g * bk + bl * gk)
                m = jnp.minimum(m, t)
            return acc + m

        return plsc.parallel_loop(0, _CH // _LANES, unroll=2,
                                  carry=acc)(body_b), acc2

    acc = jnp.zeros((_LANES,), jnp.float32)
    acc2 = jnp.zeros((_LANES,), jnp.float32)
    for ch in range(_NCH):
        s = ch % 2
        for h in handles:
            h.wait()
        if ch + 1 < _NCH:
            handles = start_chunk(ch + 1, (ch + 1) % 2)
        acc, acc2 = chunk_compute(s, acc, acc2, mbuf)

    stage_v[...] = acc + acc2
    pltpu.sync_copy(stage_v, out_px.at[wid])

    palv_h.wait()
    lanes = lax.iota(jnp.int32, _LANES)
    pv0 = palv_v[pl.ds(0 * _LANES, _LANES)]
    pv1 = palv_v[pl.ds(1 * _LANES, _LANES)]
    pv2 = palv_v[pl.ds(2 * _LANES, _LANES)]
    for qv in range(_WPB):
        @pl.when(q == qv)
        def _():
            pal_acc = jnp.zeros((_LANES,), jnp.float32)
            for j in range(qv * (_K // _WPB), (qv + 1) * (_K // _WPB)):
                dr = pv0 - pr[j]
                dg = pv1 - pg[j]
                db = pv2 - pb[j]
                d2 = dr * dr + dg * dg + db * db
                dist = d2 * _rsqrt(d2)
                mask = jnp.where(lanes > j, 1.0, 0.0).astype(jnp.float32)
                pal_acc = pal_acc + dist * mask
            stage_v[...] = pal_acc
            pltpu.sync_copy(stage_v, out_pal.at[wid])


_sc_kernel = functools.partial(
    pl.kernel,
    out_type=[
        jax.ShapeDtypeStruct((_NW, _LANES), jnp.float32),
        jax.ShapeDtypeStruct((_NW, _LANES), jnp.float32),
    ],
    mesh=plsc.VectorSubcoreMesh(core_axis_name="c", subcore_axis_name="s"),
    scratch_types=[
        pltpu.VMEM((_CH,), jnp.float32),
        pltpu.VMEM((_CH,), jnp.float32),
        pltpu.VMEM((_CH,), jnp.float32),
        pltpu.VMEM((_CH,), jnp.float32),
        pltpu.VMEM((_CH,), jnp.float32),
        pltpu.VMEM((_CH,), jnp.float32),
        pltpu.VMEM((_CH,), jnp.float32),
        pltpu.VMEM((_C * _K * _LANES,), jnp.float32),
        pltpu.VMEM((4 * _K * _LANES,), jnp.float32),
        pltpu.VMEM((_C * _LANES,), jnp.float32),
        pltpu.VMEM((_LANES,), jnp.float32),
        pltpu.SemaphoreType.DMA,
        pltpu.SemaphoreType.DMA,
        pltpu.SemaphoreType.DMA,
    ],
)(_sc_body)


@jax.jit
def kernel(palettes, images):
    palv = jnp.transpose(palettes, (0, 2, 1))
    palb = jnp.broadcast_to(palv[..., None], (_B, _C, _K, _LANES))
    palb = palb.reshape(_B, _C * _K * _LANES)
    img = images.reshape(_B * _C * _P)
    out_px, out_pal = _sc_kernel(img, palb, palv.reshape(_B, _C * _K))
    mse = jnp.sum(out_px) / (_B * _C * _P)
    pal = jnp.sum(out_pal) / (_NPAIR * _B)
    return mse - _ALPHA * pal
